# Initial kernel scaffold; baseline (speedup 1.0000x reference)
import jax
import jax.numpy as jnp
from jax.experimental import pallas as pl


def kernel(hidden_states, input_injection, graph, Wi, Wm, Wgu, Wd):
    raise NotImplementedError("write your pallas kernel here")


# speedup vs baseline: 8.0829x; 8.0829x over previous
import functools

import jax
import jax.numpy as jnp
from jax.experimental import pallas as pl
from jax.experimental.pallas import tpu as pltpu

_B, _S, _H = 2, 2048, 1024
_INTER = 2816
_TOPK = 8
_EPS = 1e-5
_LAYERS = 2

_BLK = 256


def _rms(y):
    var = jnp.mean(y * y, axis=-1, keepdims=True)
    return y * jax.lax.rsqrt(var + _EPS)


def _sparsify_body(g_ref, a_ref):
    g = g_ref[0]
    work = g
    keep = jnp.zeros(g.shape, dtype=jnp.bool_)
    col = jax.lax.broadcasted_iota(jnp.int32, g.shape, 1)
    for _ in range(_TOPK):
        m = jnp.max(work, axis=-1, keepdims=True)
        cand = jnp.where(work == m, col, _S)
        first = jnp.min(cand, axis=-1, keepdims=True)
        sel = col == first
        keep = jnp.logical_or(keep, sel)
        work = jnp.where(sel, -jnp.inf, work)
    a_ref[0] = jnp.where(keep, g, 0.0).astype(a_ref.dtype)


def _sparsify(graph):
    return pl.pallas_call(
        _sparsify_body,
        grid=(_B, _S // _BLK),
        in_specs=[pl.BlockSpec((1, _BLK, _S), lambda b, i: (b, i, 0))],
        out_specs=pl.BlockSpec((1, _BLK, _S), lambda b, i: (b, i, 0)),
        out_shape=jax.ShapeDtypeStruct((_B, _S, _S), jnp.bfloat16),
    )(graph)


def _inj_body(t_ref, wi_ref, o_ref):
    o_ref[...] = jnp.dot(t_ref[...], wi_ref[...],
                         preferred_element_type=jnp.float32)


def _inj_matmul(t2d, wi):
    n = t2d.shape[0]
    return pl.pallas_call(
        _inj_body,
        grid=(n // _BLK,),
        in_specs=[
            pl.BlockSpec((_BLK, _H), lambda i: (i, 0)),
            pl.BlockSpec((_H, _H), lambda i: (0, 0)),
        ],
        out_specs=pl.BlockSpec((_BLK, _H), lambda i: (i, 0)),
        out_shape=jax.ShapeDtypeStruct((n, _H), jnp.float32),
    )(t2d, wi)


def _msg_body(a_ref, xb_ref, x_ref, inj_ref, wm_ref, o_ref):
    t = jnp.dot(a_ref[0], xb_ref[0], preferred_element_type=jnp.float32)
    msg = jnp.dot(t.astype(jnp.bfloat16), wm_ref[...],
                  preferred_element_type=jnp.float32)
    y = x_ref[0] + inj_ref[0] + msg
    o_ref[0] = _rms(y)


def _msg_norm(a, x, inj, wm):
    xb = x.astype(jnp.bfloat16)
    return pl.pallas_call(
        _msg_body,
        grid=(_B, _S // _BLK),
        in_specs=[
            pl.BlockSpec((1, _BLK, _S), lambda b, i: (b, i, 0)),
            pl.BlockSpec((1, _S, _H), lambda b, i: (b, 0, 0)),
            pl.BlockSpec((1, _BLK, _H), lambda b, i: (b, i, 0)),
            pl.BlockSpec((1, _BLK, _H), lambda b, i: (b, i, 0)),
            pl.BlockSpec((_H, _H), lambda b, i: (0, 0)),
        ],
        out_specs=pl.BlockSpec((1, _BLK, _H), lambda b, i: (b, i, 0)),
        out_shape=jax.ShapeDtypeStruct((_B, _S, _H), jnp.float32),
    )(a, xb, x, inj, wm)


def _swiglu_body(x_ref, wgu_ref, wd_ref, o_ref):
    x = x_ref[...]
    xb = x.astype(jnp.bfloat16)
    gu = jnp.dot(xb, wgu_ref[...], preferred_element_type=jnp.float32)
    gate, up = gu[:, :_INTER], gu[:, _INTER:]
    h = (jax.nn.silu(gate) * up).astype(jnp.bfloat16)
    mlp = jnp.dot(h, wd_ref[...], preferred_element_type=jnp.float32)
    y = x + mlp
    o_ref[...] = _rms(y)


def _swiglu(x2d, wgu, wd):
    n = x2d.shape[0]
    return pl.pallas_call(
        _swiglu_body,
        grid=(n // _BLK,),
        in_specs=[
            pl.BlockSpec((_BLK, _H), lambda i: (i, 0)),
            pl.BlockSpec((_H, 2 * _INTER), lambda i: (0, 0)),
            pl.BlockSpec((_INTER, _H), lambda i: (0, 0)),
        ],
        out_specs=pl.BlockSpec((_BLK, _H), lambda i: (i, 0)),
        out_shape=jax.ShapeDtypeStruct((n, _H), jnp.float32),
    )(x2d, wgu, wd)


@jax.jit
def kernel(hidden_states, input_injection, graph, Wi, Wm, Wgu, Wd):
    a = _sparsify(graph)
    inj = _inj_matmul(
        input_injection.reshape(_B * _S, _H).astype(jnp.bfloat16),
        Wi.astype(jnp.bfloat16),
    ).reshape(_B, _S, _H)
    wm = Wm.astype(jnp.bfloat16)
    wgu = Wgu.astype(jnp.bfloat16)
    wd = Wd.astype(jnp.bfloat16)
    x = hidden_states
    for _ in range(_LAYERS):
        x = _msg_norm(a, x, inj, wm)
        x = _swiglu(x.reshape(_B * _S, _H), wgu, wd).reshape(_B, _S, _H)
    return x
